# Initial kernel scaffold; baseline (speedup 1.0000x reference)
import jax
import jax.numpy as jnp
from jax.experimental import pallas as pl


def kernel(x, edge_index, depth, W, b, fc_W, fc_b):
    raise NotImplementedError("write your pallas kernel here")


# speedup vs baseline: 10.0714x; 10.0714x over previous
import dataclasses
import functools

import jax
import jax.numpy as jnp
from jax import lax
from jax.experimental import pallas as pl
from jax.experimental.pallas import tpu as pltpu
from jax.experimental.pallas import tpu_sc as plsc

_NC = 2
_NS = 16
_LN = 16
_N = 10000
_D = 128
_NR = 10112
_RPT = _NR // _NS
_CH = 128
_CHUNKS = 79
_EP = _NC * _NS * _CHUNKS * _CH

_mesh = plsc.VectorSubcoreMesh(core_axis_name="c", subcore_axis_name="s")

_sc_params = pltpu.CompilerParams()
if "needs_layout_passes" in pltpu.CompilerParams.__dataclass_fields__:
    _sc_params = dataclasses.replace(_sc_params, needs_layout_passes=False)


def _fill(ref, value):
    width = ref.shape[1]

    @pl.loop(0, ref.shape[0])
    def _(r):
        @pl.loop(0, width // _LN)
        def _(j):
            ref[r, pl.ds(j * _LN, _LN)] = jnp.full((_LN,), value, jnp.float32)


def _zero_acc(zsrc, acc, row0):
    nz = zsrc.shape[0]

    @pl.loop(0, _RPT // nz)
    def _(j):
        pltpu.sync_copy(zsrc, acc.at[pl.ds(row0 + j * nz, nz)])

    rem = _RPT % nz
    if rem:
        pltpu.sync_copy(zsrc.at[pl.ds(0, rem)],
                        acc.at[pl.ds(row0 + (_RPT // nz) * nz, rem)])


def _sc_degree(dst_pad):
    @functools.partial(
        pl.kernel,
        out_type=jax.ShapeDtypeStruct((_NC * _NS, _NR), jnp.float32),
        mesh=_mesh,
        compiler_params=_sc_params,
        scratch_types=[
            pltpu.VMEM((_CH,), jnp.int32),
            pltpu.VMEM((_NR,), jnp.float32),
        ],
    )
    def k(dst_hbm, out_hbm, dstv, deg1):
        c = lax.axis_index("c")
        s = lax.axis_index("s")
        wid = s * _NC + c

        @pl.loop(0, _NR // _LN)
        def _(i):
            deg1[pl.ds(i * _LN, _LN)] = jnp.zeros((_LN,), jnp.float32)

        base0 = wid * (_CHUNKS * _CH)
        ones16 = jnp.ones((_LN,), jnp.float32)

        @pl.loop(0, _CHUNKS)
        def _(i):
            base = pl.multiple_of(base0 + i * _CH, _CH)
            pltpu.sync_copy(dst_hbm.at[pl.ds(base, _CH)], dstv)

            @pl.loop(0, _CH // _LN)
            def _(j):
                idx = dstv[pl.ds(j * _LN, _LN)]
                plsc.addupdate_scatter(deg1, [idx], ones16)

        pltpu.sync_copy(deg1, out_hbm.at[wid])

    return k(dst_pad)


def _sc_scatter(g, src_pad, dst_pad):
    @functools.partial(
        pl.kernel,
        out_type=jax.ShapeDtypeStruct((_NC, _NR, _D), jnp.float32),
        mesh=_mesh,
        scratch_types=[
            pltpu.VMEM((_CH,), jnp.int32),
            pltpu.VMEM((_CH,), jnp.int32),
            pltpu.VMEM((_CH, _D), jnp.float32),
            pltpu.VMEM_SHARED((_NR, _D), jnp.float32),
        ],
    )
    def k(g_hbm, src_hbm, dst_hbm, out_hbm, srcv, dstv, rows, acc):
        c = lax.axis_index("c")
        s = lax.axis_index("s")
        wid = s * _NC + c
        _fill(rows, 0.0)
        row0 = s * _RPT
        _zero_acc(rows, acc, row0)
        plsc.subcore_barrier()
        base0 = wid * (_CHUNKS * _CH)

        @pl.loop(0, _CHUNKS)
        def _(i):
            base = pl.multiple_of(base0 + i * _CH, _CH)
            pltpu.sync_copy(src_hbm.at[pl.ds(base, _CH)], srcv)
            pltpu.sync_copy(dst_hbm.at[pl.ds(base, _CH)], dstv)
            pltpu.sync_copy(g_hbm.at[srcv], rows)
            pltpu.sync_copy(rows, acc.at[dstv], add=True)

        plsc.subcore_barrier()
        pltpu.sync_copy(acc.at[pl.ds(row0, _RPT)],
                        out_hbm.at[c].at[pl.ds(row0, _RPT)])

    return k(g, src_pad, dst_pad)


def _tc_dinv(deg32):
    def body(deg_ref, out_ref):
        d = jnp.sum(deg_ref[...], axis=0, keepdims=True) + 1.0
        out_ref[...] = lax.rsqrt(d)

    return pl.pallas_call(
        body,
        out_shape=jax.ShapeDtypeStruct((1, _NR), jnp.float32),
    )(deg32)


def _tc_mm_scale(h, W, dinv):
    br = 1000

    def body(h_ref, w_ref, dinv_ref, out_ref):
        hw = jnp.dot(h_ref[...], w_ref[...],
                     preferred_element_type=jnp.float32)
        out_ref[...] = hw * dinv_ref[...]

    return pl.pallas_call(
        body,
        grid=(_N // br,),
        in_specs=[
            pl.BlockSpec((br, _D), lambda i: (i, 0)),
            pl.BlockSpec((_D, _D), lambda i: (0, 0)),
            pl.BlockSpec((br, 1), lambda i: (i, 0)),
        ],
        out_specs=pl.BlockSpec((br, _D), lambda i: (i, 0)),
        out_shape=jax.ShapeDtypeStruct((_N, _D), jnp.float32),
    )(h, W, dinv)


def _tc_merge(acc, g, dinv, b2):
    br = 1000

    def body(acc_ref, g_ref, dinv_ref, b_ref, out_ref):
        a = acc_ref[0, :, :] + acc_ref[1, :, :] + g_ref[...]
        out_ref[...] = jnp.maximum(a * dinv_ref[...] + b_ref[...], 0.0)

    return pl.pallas_call(
        body,
        grid=(_N // br,),
        in_specs=[
            pl.BlockSpec((_NC, br, _D), lambda i: (0, i, 0)),
            pl.BlockSpec((br, _D), lambda i: (i, 0)),
            pl.BlockSpec((br, 1), lambda i: (i, 0)),
            pl.BlockSpec((1, _D), lambda i: (0, 0)),
        ],
        out_specs=pl.BlockSpec((br, _D), lambda i: (i, 0)),
        out_shape=jax.ShapeDtypeStruct((_N, _D), jnp.float32),
    )(acc, g, dinv, b2)


def _tc_fc(h, fcw_row, fcb):
    def body(h_ref, w_ref, b_ref, out_ref):
        out_ref[...] = jnp.sum(h_ref[0:1, :] * w_ref[...], axis=1,
                               keepdims=True) + b_ref[...]

    return pl.pallas_call(
        body,
        grid=(1,),
        in_specs=[
            pl.BlockSpec((8, _D), lambda i: (0, 0)),
            pl.BlockSpec((1, _D), lambda i: (0, 0)),
            pl.BlockSpec((1, 1), lambda i: (0, 0)),
        ],
        out_specs=pl.BlockSpec((1, 1), lambda i: (0, 0)),
        out_shape=jax.ShapeDtypeStruct((1, 1), jnp.float32),
    )(h, fcw_row, fcb)


def kernel(x, edge_index, depth, W, b, fc_W, fc_b):
    src = edge_index[0]
    dst = edge_index[1]
    pad = _EP - src.shape[0]
    src_p = jnp.concatenate([src, jnp.zeros((pad,), jnp.int32)])
    dst_p = jnp.concatenate([dst, jnp.full((pad,), _N, jnp.int32)])

    deg32 = _sc_degree(dst_p)
    dinv = _tc_dinv(deg32).reshape(_NR, 1)
    b2 = b.reshape(1, _D)

    def body(_, h):
        g = _tc_mm_scale(h, W, dinv)
        acc = _sc_scatter(g, src_p, dst_p)
        return _tc_merge(acc, g, dinv, b2)

    h = lax.fori_loop(0, depth, body, x)
    out = _tc_fc(h, fc_W.reshape(1, _D), fc_b.reshape(1, 1))
    return out.reshape(1)
